# Initial kernel scaffold; baseline (speedup 1.0000x reference)
import jax
import jax.numpy as jnp
from jax.experimental import pallas as pl


def kernel(svg_path, svg_path_mask, edge_index, type_embed, coor_embed, W_in, b_in, W1p, b1p, W1l, b1l, W1r, W2p, b2p, W2l, b2l, W2r):
    raise NotImplementedError("write your pallas kernel here")


# speedup vs baseline: 1.0006x; 1.0006x over previous
import functools

import jax
import jax.numpy as jnp
from jax.experimental import pallas as pl
from jax.experimental.pallas import tpu as pltpu

_B, _N, _C = 2, 1250, 4
_D = 512
_NODES = _B * _N * _C
_E = 160000
_BM = 1000
_TPAD = 256


def _embed_matmul_body(idx_ref, table_ref, w_ref, b_ref, out_ref):
    idx = idx_ref[0, 0, :]
    onehot = (idx[:, None] == jax.lax.broadcasted_iota(jnp.int32, (_BM, _TPAD), 1)).astype(jnp.float32)
    embed = jnp.dot(onehot, table_ref[...], preferred_element_type=jnp.float32)
    out_ref[...] = jnp.dot(embed, w_ref[...], preferred_element_type=jnp.float32) + b_ref[...]


def _embed_matmul(idx_all, table, w, b):
    grid = _NODES // _BM
    idx3 = idx_all.reshape(grid, 1, _BM)
    return pl.pallas_call(
        _embed_matmul_body,
        grid=(grid,),
        in_specs=[
            pl.BlockSpec((1, 1, _BM), lambda i: (i, 0, 0)),
            pl.BlockSpec((_TPAD, _D), lambda i: (0, 0)),
            pl.BlockSpec((_D, _D), lambda i: (0, 0)),
            pl.BlockSpec((1, _D), lambda i: (0, 0)),
        ],
        out_specs=pl.BlockSpec((_BM, _D), lambda i: (i, 0)),
        out_shape=jax.ShapeDtypeStruct((_NODES, _D), jnp.float32),
    )(idx3, table, w, b)


def _relu_matmul_body(x_ref, w_ref, b_ref, out_ref):
    out_ref[...] = jax.nn.relu(
        jnp.dot(x_ref[...], w_ref[...], preferred_element_type=jnp.float32) + b_ref[...]
    )


def _relu_matmul(x, w, b):
    grid = _NODES // _BM
    return pl.pallas_call(
        _relu_matmul_body,
        grid=(grid,),
        in_specs=[
            pl.BlockSpec((_BM, _D), lambda i: (i, 0)),
            pl.BlockSpec((_D, _D), lambda i: (0, 0)),
            pl.BlockSpec((1, _D), lambda i: (0, 0)),
        ],
        out_specs=pl.BlockSpec((_BM, _D), lambda i: (i, 0)),
        out_shape=jax.ShapeDtypeStruct((_NODES, _D), jnp.float32),
    )(x, w, b.reshape(1, _D))


def _out_norm_body(mean_ref, xp_ref, wl_ref, bl_ref, wr_ref, out_ref):
    out = (
        jnp.dot(mean_ref[...], wl_ref[...], preferred_element_type=jnp.float32)
        + bl_ref[...]
        + jnp.dot(xp_ref[...], wr_ref[...], preferred_element_type=jnp.float32)
    )
    ssq = jnp.sum(out * out, axis=-1, keepdims=True)
    norm = jnp.sqrt(ssq)
    out_ref[...] = out / jnp.maximum(norm, 1e-12)


def _out_norm(mean, xp, wl, bl, wr):
    grid = _NODES // _BM
    return pl.pallas_call(
        _out_norm_body,
        grid=(grid,),
        in_specs=[
            pl.BlockSpec((_BM, _D), lambda i: (i, 0)),
            pl.BlockSpec((_BM, _D), lambda i: (i, 0)),
            pl.BlockSpec((_D, _D), lambda i: (0, 0)),
            pl.BlockSpec((1, _D), lambda i: (0, 0)),
            pl.BlockSpec((_D, _D), lambda i: (0, 0)),
        ],
        out_specs=pl.BlockSpec((_BM, _D), lambda i: (i, 0)),
        out_shape=jax.ShapeDtypeStruct((_NODES, _D), jnp.float32),
    )(mean, xp, wl, bl.reshape(1, _D), wr)


def _segment_mean(xp, src, dst, cnt_inv):
    msgs = jnp.take(xp, src, axis=0)
    summed = jax.ops.segment_sum(msgs, dst, num_segments=_NODES)
    return summed * cnt_inv


def kernel(svg_path, svg_path_mask, edge_index, type_embed, coor_embed, W_in, b_in,
           W1p, b1p, W1l, b1l, W1r, W2p, b2p, W2l, b2l, W2r):
    svg = jnp.where(svg_path_mask, svg_path, 0)
    cmd_idx = svg[:, :, 0]
    coor_idx = svg[:, :, 1:] + 3
    idx_all = jnp.concatenate(
        [cmd_idx.reshape(_B, _N), coor_idx.reshape(_B, _N * (_C - 1))], axis=1
    ).reshape(_NODES)
    table = jnp.concatenate(
        [type_embed, coor_embed,
         jnp.zeros((_TPAD - 3 - 200, _D), jnp.float32)], axis=0
    )
    src = edge_index[0]
    dst = edge_index[1]
    cnt = jax.ops.segment_sum(jnp.ones((_E,), jnp.float32), dst, num_segments=_NODES)
    cnt_inv = (1.0 / jnp.maximum(cnt, 1.0))[:, None]

    x = _embed_matmul(idx_all, table, W_in, b_in.reshape(1, _D))

    xp1 = _relu_matmul(x, W1p, b1p)
    mean1 = _segment_mean(xp1, src, dst, cnt_inv)
    x1 = _out_norm(mean1, xp1, W1l, b1l, W1r)

    xp2 = _relu_matmul(x1, W2p, b2p)
    mean2 = _segment_mean(xp2, src, dst, cnt_inv)
    x2 = _out_norm(mean2, xp2, W2l, b2l, W2r)
    return x2


# speedup vs baseline: 2.0432x; 2.0419x over previous
import jax
import jax.numpy as jnp
from jax import lax
from jax.experimental import pallas as pl
from jax.experimental.pallas import tpu as pltpu
from jax.experimental.pallas import tpu_sc as plsc

_B, _N, _C = 2, 1250, 4
_D = 512
_NODES = _B * _N * _C
_E = 160000
_BM = 1000
_TPAD = 256

_NC = 2
_NS = 16
_NW = _NC * _NS
_EPT = _E // _NW
_NCHUNK = _EPT // 16
_NBKT = 10
_BKT = _NODES // _NBKT
_ACCR = 1024
_TRASH = 1016
_GCAP = 5248
_HISTR = 80


def _embed_matmul_body(idx_ref, table_ref, w_ref, b_ref, out_ref):
    idx = idx_ref[0, 0, :]
    onehot = (idx[:, None] == lax.broadcasted_iota(jnp.int32, (_BM, _TPAD), 1)).astype(jnp.float32)
    embed = jnp.dot(onehot, table_ref[...], preferred_element_type=jnp.float32)
    out_ref[...] = jnp.dot(embed, w_ref[...], preferred_element_type=jnp.float32) + b_ref[...]


def _embed_matmul(idx_all, table, w, b):
    grid = _NODES // _BM
    idx3 = idx_all.reshape(grid, 1, _BM)
    return pl.pallas_call(
        _embed_matmul_body,
        grid=(grid,),
        in_specs=[
            pl.BlockSpec((1, 1, _BM), lambda i: (i, 0, 0)),
            pl.BlockSpec((_TPAD, _D), lambda i: (0, 0)),
            pl.BlockSpec((_D, _D), lambda i: (0, 0)),
            pl.BlockSpec((1, _D), lambda i: (0, 0)),
        ],
        out_specs=pl.BlockSpec((_BM, _D), lambda i: (i, 0)),
        out_shape=jax.ShapeDtypeStruct((_NODES, _D), jnp.float32),
    )(idx3, table, w, b)


def _relu_matmul_body(x_ref, w_ref, b_ref, out_ref):
    out_ref[...] = jax.nn.relu(
        jnp.dot(x_ref[...], w_ref[...], preferred_element_type=jnp.float32) + b_ref[...]
    )


def _relu_matmul(x, w, b):
    grid = _NODES // _BM
    return pl.pallas_call(
        _relu_matmul_body,
        grid=(grid,),
        in_specs=[
            pl.BlockSpec((_BM, _D), lambda i: (i, 0)),
            pl.BlockSpec((_D, _D), lambda i: (0, 0)),
            pl.BlockSpec((1, _D), lambda i: (0, 0)),
        ],
        out_specs=pl.BlockSpec((_BM, _D), lambda i: (i, 0)),
        out_shape=jax.ShapeDtypeStruct((_NODES, _D), jnp.float32),
    )(x, w, b.reshape(1, _D))


def _out_norm_body(p0_ref, p1_ref, cinv_ref, xp_ref, wl_ref, bl_ref, wr_ref, out_ref):
    mean = (p0_ref[0] + p1_ref[0]) * cinv_ref[...]
    out = (
        jnp.dot(mean, wl_ref[...], preferred_element_type=jnp.float32)
        + bl_ref[...]
        + jnp.dot(xp_ref[...], wr_ref[...], preferred_element_type=jnp.float32)
    )
    ssq = jnp.sum(out * out, axis=-1, keepdims=True)
    out_ref[...] = out / jnp.maximum(jnp.sqrt(ssq), 1e-12)


def _out_norm(parts, cnt_inv, xp, wl, bl, wr):
    grid = _NODES // _BM
    return pl.pallas_call(
        _out_norm_body,
        grid=(grid,),
        in_specs=[
            pl.BlockSpec((1, _BM, _D), lambda i: (0, i, 0)),
            pl.BlockSpec((1, _BM, _D), lambda i: (1, i, 0)),
            pl.BlockSpec((_BM, 1), lambda i: (i, 0)),
            pl.BlockSpec((_BM, _D), lambda i: (i, 0)),
            pl.BlockSpec((_D, _D), lambda i: (0, 0)),
            pl.BlockSpec((1, _D), lambda i: (0, 0)),
            pl.BlockSpec((_D, _D), lambda i: (0, 0)),
        ],
        out_specs=pl.BlockSpec((_BM, _D), lambda i: (i, 0)),
        out_shape=jax.ShapeDtypeStruct((_NODES, _D), jnp.float32),
    )(parts, parts, cnt_inv, xp, wl, bl.reshape(1, _D), wr)


_SC_MESH = plsc.VectorSubcoreMesh(core_axis_name="c", subcore_axis_name="s")
_SC_PARAMS = pltpu.CompilerParams(use_tc_tiling_on_sc=False,
                                  needs_layout_passes=False)


def _cnt_body(dst_hbm, out_hbm, dst_v, hist_v, cnt_sh, sem):
    c = lax.axis_index("c")
    s = lax.axis_index("s")
    wid = s * _NC + c
    pltpu.sync_copy(dst_hbm.at[pl.ds(wid * _EPT, _EPT)], dst_v.at[pl.ds(0, _EPT)])

    zero16 = jnp.zeros((16,), jnp.float32)

    def zero_hist(t, carry):
        hist_v[t // 8, pl.ds((t % 8) * 16, 16)] = zero16
        return carry
    lax.fori_loop(0, _HISTR * 8, zero_hist, 0)

    @pl.when(s < 10)
    def _zero_sh():
        pltpu.sync_copy(hist_v.at[pl.ds(s * 8, 8)], cnt_sh.at[pl.ds(s * 8, 8)])
    plsc.subcore_barrier()

    ones = jnp.ones((16,), jnp.float32)
    lanes = lax.iota(jnp.int32, 16)

    def count(i, carry):
        nlanes = jnp.where(i == _NCHUNK, 8, 16)
        m = lanes < nlanes
        d = dst_v[pl.ds(i * 16, 16)]
        plsc.addupdate_scatter(hist_v, [lax.shift_right_logical(d, 7), d & 127],
                               ones, mask=m)
        return carry
    lax.fori_loop(0, _NCHUNK + 1, count, 0)

    for k in range(_HISTR // 16):
        idxv = lax.iota(jnp.int32, 16) + k * 16
        pltpu.sync_copy(hist_v.at[pl.ds(k * 16, 16)], cnt_sh.at[idxv], add=True)
    plsc.subcore_barrier()

    @pl.when(s < 10)
    def _writeback():
        pltpu.sync_copy(cnt_sh.at[pl.ds(s * 8, 8)], out_hbm.at[c, pl.ds(s * 8, 8)])


def _sc_counts(dst):
    return pl.kernel(
        _cnt_body,
        out_type=jax.ShapeDtypeStruct((_NC, _HISTR, 128), jnp.float32),
        mesh=_SC_MESH,
        compiler_params=_SC_PARAMS,
        scratch_types=[
            pltpu.VMEM((_EPT + 16,), jnp.int32),
            pltpu.VMEM((_HISTR, 128), jnp.float32),
            pltpu.VMEM_SHARED((_HISTR, 128), jnp.float32),
            pltpu.SemaphoreType.DMA,
        ],
    )(dst)


def _segsum_body(xp_hbm, src_hbm, dst_hbm, zrows_hbm, out_hbm,
                 src_v, dst_v, gsrc_v, gdst_v, rows_v, zbuf_v, acc_sh, sem):
    c = lax.axis_index("c")
    s = lax.axis_index("s")
    wid = s * _NC + c
    pltpu.sync_copy(src_hbm.at[pl.ds(wid * _EPT, _EPT)], src_v.at[pl.ds(0, _EPT)])
    pltpu.sync_copy(dst_hbm.at[pl.ds(wid * _EPT, _EPT)], dst_v.at[pl.ds(0, _EPT)])
    pltpu.sync_copy(zrows_hbm, zbuf_v)

    lanes = lax.iota(jnp.int32, 16)
    trash16 = jnp.full((16,), _TRASH, jnp.int32)
    zero16i = jnp.zeros((16,), jnp.int32)

    for q in range(_NBKT):
        lo = q * _BKT

        for t in range(4):
            pltpu.sync_copy(zbuf_v, acc_sh.at[pl.ds(s * 64 + t * 16, 16)])
        plsc.subcore_barrier()

        def filt(i, n):
            nlanes = jnp.where(i == _NCHUNK, 8, 16)
            m = lanes < nlanes
            d = dst_v[pl.ds(i * 16, 16)]
            sv = src_v[pl.ds(i * 16, 16)]
            inb = m & (d >= lo) & (d < lo + _BKT)
            plsc.store_compressed(gsrc_v.at[pl.ds(n, 16)], sv, mask=inb)
            plsc.store_compressed(gdst_v.at[pl.ds(n, 16)], d - lo, mask=inb)
            return n + jnp.sum(inb.astype(jnp.int32))
        n = lax.fori_loop(0, _NCHUNK + 1, filt, 0)

        for t in range(8):
            gsrc_v[pl.ds(n + t * 16, 16)] = zero16i
            gdst_v[pl.ds(n + t * 16, 16)] = trash16

        def chunk(j, carry):
            pltpu.async_copy(xp_hbm.at[gsrc_v.at[pl.ds(j * 128, 128)]],
                             rows_v, sem).wait()
            for k in range(8):
                dv = gdst_v[pl.ds(j * 128 + k * 16, 16)]
                pltpu.sync_copy(rows_v.at[pl.ds(k * 16, 16)], acc_sh.at[dv], add=True)
            return carry
        lax.fori_loop(0, (n + 127) // 128, chunk, 0)
        plsc.subcore_barrier()

        @pl.when(s < 15)
        def _wb():
            pltpu.sync_copy(acc_sh.at[pl.ds(s * 64, 64)],
                            out_hbm.at[c, pl.ds(lo + s * 64, 64)])

        @pl.when(s == 15)
        def _wb_tail():
            pltpu.sync_copy(acc_sh.at[pl.ds(960, 40)],
                            out_hbm.at[c, pl.ds(lo + 960, 40)])
        plsc.subcore_barrier()


def _sc_segsum(xp, src, dst, zrows):
    return pl.kernel(
        _segsum_body,
        out_type=jax.ShapeDtypeStruct((_NC, _NODES, _D), jnp.float32),
        mesh=_SC_MESH,
        compiler_params=_SC_PARAMS,
        scratch_types=[
            pltpu.VMEM((_EPT + 16,), jnp.int32),
            pltpu.VMEM((_EPT + 16,), jnp.int32),
            pltpu.VMEM((_GCAP,), jnp.int32),
            pltpu.VMEM((_GCAP,), jnp.int32),
            pltpu.VMEM((128, _D), jnp.float32),
            pltpu.VMEM((16, _D), jnp.float32),
            pltpu.VMEM_SHARED((_ACCR, _D), jnp.float32),
            pltpu.SemaphoreType.DMA,
        ],
    )(xp, src, dst, zrows)


def kernel(svg_path, svg_path_mask, edge_index, type_embed, coor_embed, W_in, b_in,
           W1p, b1p, W1l, b1l, W1r, W2p, b2p, W2l, b2l, W2r):
    svg = jnp.where(svg_path_mask, svg_path, 0)
    cmd_idx = svg[:, :, 0]
    coor_idx = svg[:, :, 1:] + 3
    idx_all = jnp.concatenate(
        [cmd_idx.reshape(_B, _N), coor_idx.reshape(_B, _N * (_C - 1))], axis=1
    ).reshape(_NODES)
    table = jnp.concatenate(
        [type_embed, coor_embed,
         jnp.zeros((_TPAD - 3 - 200, _D), jnp.float32)], axis=0
    )
    src = edge_index[0]
    dst = edge_index[1]
    zrows = jnp.zeros((16, _D), jnp.float32)

    cntp = _sc_counts(dst)
    cnt = (cntp[0] + cntp[1]).reshape(_HISTR * 128)[:_NODES]
    cnt_inv = (1.0 / jnp.maximum(cnt, 1.0)).reshape(_NODES, 1)

    x = _embed_matmul(idx_all, table, W_in, b_in.reshape(1, _D))

    xp1 = _relu_matmul(x, W1p, b1p)
    parts1 = _sc_segsum(xp1, src, dst, zrows)
    x1 = _out_norm(parts1, cnt_inv, xp1, W1l, b1l, W1r)

    xp2 = _relu_matmul(x1, W2p, b2p)
    parts2 = _sc_segsum(xp2, src, dst, zrows)
    x2 = _out_norm(parts2, cnt_inv, xp2, W2l, b2l, W2r)
    return x2


# speedup vs baseline: 2.5570x; 1.2515x over previous
import jax
import jax.numpy as jnp
from jax import lax
from jax.experimental import pallas as pl
from jax.experimental.pallas import tpu as pltpu
from jax.experimental.pallas import tpu_sc as plsc

_B, _N, _C = 2, 1250, 4
_D = 512
_NODES = _B * _N * _C
_E = 160000
_BM = 1000
_TPAD = 256

_NC = 2
_NS = 16
_NW = _NC * _NS
_EPT = _E // _NW
_NCHUNK = _EPT // 16
_NBKT = 10
_BKT = _NODES // _NBKT
_ACCR = 1024
_TRASH = 1016
_GCAP = 5248
_HISTR = 80


def _embed_matmul_body(idx_ref, table_ref, w_ref, b_ref, out_ref):
    idx = idx_ref[0, 0, :]
    onehot = (idx[:, None] == lax.broadcasted_iota(jnp.int32, (_BM, _TPAD), 1)).astype(jnp.float32)
    embed = jnp.dot(onehot, table_ref[...], preferred_element_type=jnp.float32)
    out_ref[...] = jnp.dot(embed, w_ref[...], preferred_element_type=jnp.float32) + b_ref[...]


def _embed_matmul(idx_all, table, w, b):
    grid = _NODES // _BM
    idx3 = idx_all.reshape(grid, 1, _BM)
    return pl.pallas_call(
        _embed_matmul_body,
        grid=(grid,),
        in_specs=[
            pl.BlockSpec((1, 1, _BM), lambda i: (i, 0, 0)),
            pl.BlockSpec((_TPAD, _D), lambda i: (0, 0)),
            pl.BlockSpec((_D, _D), lambda i: (0, 0)),
            pl.BlockSpec((1, _D), lambda i: (0, 0)),
        ],
        out_specs=pl.BlockSpec((_BM, _D), lambda i: (i, 0)),
        out_shape=jax.ShapeDtypeStruct((_NODES, _D), jnp.float32),
    )(idx3, table, w, b)


def _relu_matmul_body(x_ref, w_ref, b_ref, out_ref):
    out_ref[...] = jax.nn.relu(
        jnp.dot(x_ref[...], w_ref[...], preferred_element_type=jnp.float32) + b_ref[...]
    )


def _relu_matmul(x, w, b):
    grid = _NODES // _BM
    return pl.pallas_call(
        _relu_matmul_body,
        grid=(grid,),
        in_specs=[
            pl.BlockSpec((_BM, _D), lambda i: (i, 0)),
            pl.BlockSpec((_D, _D), lambda i: (0, 0)),
            pl.BlockSpec((1, _D), lambda i: (0, 0)),
        ],
        out_specs=pl.BlockSpec((_BM, _D), lambda i: (i, 0)),
        out_shape=jax.ShapeDtypeStruct((_NODES, _D), jnp.float32),
    )(x, w, b.reshape(1, _D))


def _out_norm_body(p0_ref, p1_ref, cinv_ref, xp_ref, wl_ref, bl_ref, wr_ref, out_ref):
    mean = (p0_ref[0] + p1_ref[0]) * cinv_ref[...]
    out = (
        jnp.dot(mean, wl_ref[...], preferred_element_type=jnp.float32)
        + bl_ref[...]
        + jnp.dot(xp_ref[...], wr_ref[...], preferred_element_type=jnp.float32)
    )
    ssq = jnp.sum(out * out, axis=-1, keepdims=True)
    out_ref[...] = out / jnp.maximum(jnp.sqrt(ssq), 1e-12)


def _out_norm(parts, cnt_inv, xp, wl, bl, wr):
    grid = _NODES // _BM
    return pl.pallas_call(
        _out_norm_body,
        grid=(grid,),
        in_specs=[
            pl.BlockSpec((1, _BM, _D), lambda i: (0, i, 0)),
            pl.BlockSpec((1, _BM, _D), lambda i: (1, i, 0)),
            pl.BlockSpec((_BM, 1), lambda i: (i, 0)),
            pl.BlockSpec((_BM, _D), lambda i: (i, 0)),
            pl.BlockSpec((_D, _D), lambda i: (0, 0)),
            pl.BlockSpec((1, _D), lambda i: (0, 0)),
            pl.BlockSpec((_D, _D), lambda i: (0, 0)),
        ],
        out_specs=pl.BlockSpec((_BM, _D), lambda i: (i, 0)),
        out_shape=jax.ShapeDtypeStruct((_NODES, _D), jnp.float32),
    )(parts, parts, cnt_inv, xp, wl, bl.reshape(1, _D), wr)


_SC_MESH = plsc.VectorSubcoreMesh(core_axis_name="c", subcore_axis_name="s")
_SC_PARAMS = pltpu.CompilerParams(use_tc_tiling_on_sc=False,
                                  needs_layout_passes=False)


def _cnt_body(dst_hbm, out_hbm, dst_v, hist_v, cnt_sh, sem):
    c = lax.axis_index("c")
    s = lax.axis_index("s")
    wid = s * _NC + c
    pltpu.sync_copy(dst_hbm.at[pl.ds(wid * _EPT, _EPT)], dst_v.at[pl.ds(0, _EPT)])

    zero16 = jnp.zeros((16,), jnp.float32)

    def zero_hist(t, carry):
        hist_v[t // 8, pl.ds((t % 8) * 16, 16)] = zero16
        return carry
    lax.fori_loop(0, _HISTR * 8, zero_hist, 0)

    @pl.when(s < 10)
    def _zero_sh():
        pltpu.sync_copy(hist_v.at[pl.ds(s * 8, 8)], cnt_sh.at[pl.ds(s * 8, 8)])
    plsc.subcore_barrier()

    ones = jnp.ones((16,), jnp.float32)
    lanes = lax.iota(jnp.int32, 16)

    def count(i, carry):
        nlanes = jnp.where(i == _NCHUNK, 8, 16)
        m = lanes < nlanes
        d = dst_v[pl.ds(i * 16, 16)]
        plsc.addupdate_scatter(hist_v, [lax.shift_right_logical(d, 7), d & 127],
                               ones, mask=m)
        return carry
    lax.fori_loop(0, _NCHUNK + 1, count, 0)

    for k in range(_HISTR // 16):
        idxv = lax.iota(jnp.int32, 16) + k * 16
        pltpu.sync_copy(hist_v.at[pl.ds(k * 16, 16)], cnt_sh.at[idxv], add=True)
    plsc.subcore_barrier()

    @pl.when(s < 10)
    def _writeback():
        pltpu.sync_copy(cnt_sh.at[pl.ds(s * 8, 8)], out_hbm.at[c, pl.ds(s * 8, 8)])


def _sc_counts(dst):
    return pl.kernel(
        _cnt_body,
        out_type=jax.ShapeDtypeStruct((_NC, _HISTR, 128), jnp.float32),
        mesh=_SC_MESH,
        compiler_params=_SC_PARAMS,
        scratch_types=[
            pltpu.VMEM((_EPT + 16,), jnp.int32),
            pltpu.VMEM((_HISTR, 128), jnp.float32),
            pltpu.VMEM_SHARED((_HISTR, 128), jnp.float32),
            pltpu.SemaphoreType.DMA,
        ],
    )(dst)


def _segsum_body(xp_hbm, src_hbm, dst_hbm, zrows_hbm, out_hbm,
                 src_v, dst_v, gsrc_v, gdst_v, rows_a, rows_b, didx_a, didx_b,
                 zbuf_v, acc_sh, sem_a, sem_b):
    c = lax.axis_index("c")
    s = lax.axis_index("s")
    wid = s * _NC + c
    pltpu.sync_copy(src_hbm.at[pl.ds(wid * _EPT, _EPT)], src_v.at[pl.ds(0, _EPT)])
    pltpu.sync_copy(dst_hbm.at[pl.ds(wid * _EPT, _EPT)], dst_v.at[pl.ds(0, _EPT)])
    pltpu.sync_copy(zrows_hbm, zbuf_v)

    lanes = lax.iota(jnp.int32, 16)
    trash16 = jnp.full((16,), _TRASH, jnp.int32)
    zero16i = jnp.zeros((16,), jnp.int32)

    for q in range(_NBKT):
        lo = q * _BKT

        for t in range(4):
            pltpu.sync_copy(zbuf_v, acc_sh.at[pl.ds(s * 64 + t * 16, 16)])
        plsc.subcore_barrier()

        def filt(i, n):
            nlanes = jnp.where(i == _NCHUNK, 8, 16)
            m = lanes < nlanes
            d = dst_v[pl.ds(i * 16, 16)]
            sv = src_v[pl.ds(i * 16, 16)]
            inb = m & (d >= lo) & (d < lo + _BKT)
            plsc.store_compressed(gsrc_v.at[pl.ds(n, 16)], sv, mask=inb)
            plsc.store_compressed(gdst_v.at[pl.ds(n, 16)], d - lo, mask=inb)
            return n + jnp.sum(inb.astype(jnp.int32))
        n = lax.fori_loop(0, _NCHUNK + 1, filt, 0)

        for t in range(5):
            gsrc_v[pl.ds(n + t * 16, 16)] = zero16i
            gdst_v[pl.ds(n + t * 16, 16)] = trash16

        nch = (n + 63) // 64

        @pl.when(nch > 0)
        def _prime():
            pltpu.async_copy(xp_hbm.at[gsrc_v.at[pl.ds(0, 64)]], rows_a, sem_a)

        def chunk(j, carry):
            nxt = j + 1

            @pl.when(j % 2 == 0)
            def _even():
                pltpu.make_async_copy(xp_hbm.at[gsrc_v.at[pl.ds(j * 64, 64)]],
                                      rows_a, sem_a).wait()

                @pl.when(nxt < nch)
                def _issue():
                    pltpu.async_copy(xp_hbm.at[gsrc_v.at[pl.ds(nxt * 64, 64)]],
                                     rows_b, sem_b)
                for k in range(4):
                    didx_a[pl.ds(k * 16, 16)] = gdst_v[pl.ds(j * 64 + k * 16, 16)]
                pltpu.sync_copy(rows_a, acc_sh.at[didx_a], add=True)

            @pl.when(j % 2 == 1)
            def _odd():
                pltpu.make_async_copy(xp_hbm.at[gsrc_v.at[pl.ds(j * 64, 64)]],
                                      rows_b, sem_b).wait()

                @pl.when(nxt < nch)
                def _issue():
                    pltpu.async_copy(xp_hbm.at[gsrc_v.at[pl.ds(nxt * 64, 64)]],
                                     rows_a, sem_a)
                for k in range(4):
                    didx_b[pl.ds(k * 16, 16)] = gdst_v[pl.ds(j * 64 + k * 16, 16)]
                pltpu.sync_copy(rows_b, acc_sh.at[didx_b], add=True)
            return carry
        lax.fori_loop(0, nch, chunk, 0)
        plsc.subcore_barrier()

        @pl.when(s < 15)
        def _wb():
            pltpu.sync_copy(acc_sh.at[pl.ds(s * 64, 64)],
                            out_hbm.at[c, pl.ds(lo + s * 64, 64)])

        @pl.when(s == 15)
        def _wb_tail():
            pltpu.sync_copy(acc_sh.at[pl.ds(960, 40)],
                            out_hbm.at[c, pl.ds(lo + 960, 40)])
        plsc.subcore_barrier()


def _sc_segsum(xp, src, dst, zrows):
    return pl.kernel(
        _segsum_body,
        out_type=jax.ShapeDtypeStruct((_NC, _NODES, _D), jnp.float32),
        mesh=_SC_MESH,
        compiler_params=_SC_PARAMS,
        scratch_types=[
            pltpu.VMEM((_EPT + 16,), jnp.int32),
            pltpu.VMEM((_EPT + 16,), jnp.int32),
            pltpu.VMEM((_GCAP,), jnp.int32),
            pltpu.VMEM((_GCAP,), jnp.int32),
            pltpu.VMEM((64, _D), jnp.float32),
            pltpu.VMEM((64, _D), jnp.float32),
            pltpu.VMEM((64,), jnp.int32),
            pltpu.VMEM((64,), jnp.int32),
            pltpu.VMEM((16, _D), jnp.float32),
            pltpu.VMEM_SHARED((_ACCR, _D), jnp.float32),
            pltpu.SemaphoreType.DMA,
            pltpu.SemaphoreType.DMA,
        ],
    )(xp, src, dst, zrows)


def kernel(svg_path, svg_path_mask, edge_index, type_embed, coor_embed, W_in, b_in,
           W1p, b1p, W1l, b1l, W1r, W2p, b2p, W2l, b2l, W2r):
    svg = jnp.where(svg_path_mask, svg_path, 0)
    cmd_idx = svg[:, :, 0]
    coor_idx = svg[:, :, 1:] + 3
    idx_all = jnp.concatenate(
        [cmd_idx.reshape(_B, _N), coor_idx.reshape(_B, _N * (_C - 1))], axis=1
    ).reshape(_NODES)
    table = jnp.concatenate(
        [type_embed, coor_embed,
         jnp.zeros((_TPAD - 3 - 200, _D), jnp.float32)], axis=0
    )
    src = edge_index[0]
    dst = edge_index[1]
    zrows = jnp.zeros((16, _D), jnp.float32)

    cntp = _sc_counts(dst)
    cnt = (cntp[0] + cntp[1]).reshape(_HISTR * 128)[:_NODES]
    cnt_inv = (1.0 / jnp.maximum(cnt, 1.0)).reshape(_NODES, 1)

    x = _embed_matmul(idx_all, table, W_in, b_in.reshape(1, _D))

    xp1 = _relu_matmul(x, W1p, b1p)
    parts1 = _sc_segsum(xp1, src, dst, zrows)
    x1 = _out_norm(parts1, cnt_inv, xp1, W1l, b1l, W1r)

    xp2 = _relu_matmul(x1, W2p, b2p)
    parts2 = _sc_segsum(xp2, src, dst, zrows)
    x2 = _out_norm(parts2, cnt_inv, xp2, W2l, b2l, W2r)
    return x2


# speedup vs baseline: 2.7838x; 1.0887x over previous
import jax
import jax.numpy as jnp
from jax import lax
from jax.experimental import pallas as pl
from jax.experimental.pallas import tpu as pltpu
from jax.experimental.pallas import tpu_sc as plsc

_B, _N, _C = 2, 1250, 4
_D = 512
_NODES = _B * _N * _C
_E = 160000
_BM = 1000
_TPAD = 256

_NC = 2
_NS = 16
_NW = _NC * _NS
_EPT = _E // _NW
_NCHUNK = _EPT // 16
_NBKT = 10
_BKT = _NODES // _NBKT
_ACCR = 1024
_TRASH = 1016
_ROWS = 64
_CAP = 5120
_HISTR = 80


def _embed_matmul_body(idx_ref, table_ref, w_ref, b_ref, out_ref):
    idx = idx_ref[0, 0, :]
    onehot = (idx[:, None] == lax.broadcasted_iota(jnp.int32, (_BM, _TPAD), 1)).astype(jnp.float32)
    embed = jnp.dot(onehot, table_ref[...], preferred_element_type=jnp.float32)
    out_ref[...] = jnp.dot(embed, w_ref[...], preferred_element_type=jnp.float32) + b_ref[...]


def _embed_matmul(idx_all, table, w, b):
    grid = _NODES // _BM
    idx3 = idx_all.reshape(grid, 1, _BM)
    return pl.pallas_call(
        _embed_matmul_body,
        grid=(grid,),
        in_specs=[
            pl.BlockSpec((1, 1, _BM), lambda i: (i, 0, 0)),
            pl.BlockSpec((_TPAD, _D), lambda i: (0, 0)),
            pl.BlockSpec((_D, _D), lambda i: (0, 0)),
            pl.BlockSpec((1, _D), lambda i: (0, 0)),
        ],
        out_specs=pl.BlockSpec((_BM, _D), lambda i: (i, 0)),
        out_shape=jax.ShapeDtypeStruct((_NODES, _D), jnp.float32),
    )(idx3, table, w, b)


def _relu_matmul_body(x_ref, w_ref, b_ref, out_ref):
    out_ref[...] = jax.nn.relu(
        jnp.dot(x_ref[...], w_ref[...], preferred_element_type=jnp.float32) + b_ref[...]
    )


def _relu_matmul(x, w, b):
    grid = _NODES // _BM
    return pl.pallas_call(
        _relu_matmul_body,
        grid=(grid,),
        in_specs=[
            pl.BlockSpec((_BM, _D), lambda i: (i, 0)),
            pl.BlockSpec((_D, _D), lambda i: (0, 0)),
            pl.BlockSpec((1, _D), lambda i: (0, 0)),
        ],
        out_specs=pl.BlockSpec((_BM, _D), lambda i: (i, 0)),
        out_shape=jax.ShapeDtypeStruct((_NODES, _D), jnp.float32),
    )(x, w, b.reshape(1, _D))


def _matmul_body(x_ref, w_ref, out_ref):
    out_ref[...] = jnp.dot(x_ref[...], w_ref[...], preferred_element_type=jnp.float32)


def _matmul(x, w):
    grid = _NODES // _BM
    return pl.pallas_call(
        _matmul_body,
        grid=(grid,),
        in_specs=[
            pl.BlockSpec((_BM, _D), lambda i: (i, 0)),
            pl.BlockSpec((_D, _D), lambda i: (0, 0)),
        ],
        out_specs=pl.BlockSpec((_BM, _D), lambda i: (i, 0)),
        out_shape=jax.ShapeDtypeStruct((_NODES, _D), jnp.float32),
    )(x, w)


def _out_norm_body(p0_ref, p1_ref, cinv_ref, yr_ref, wl_ref, bl_ref, out_ref):
    mean = (p0_ref[0] + p1_ref[0]) * cinv_ref[...]
    out = (
        jnp.dot(mean, wl_ref[...], preferred_element_type=jnp.float32)
        + bl_ref[...]
        + yr_ref[...]
    )
    ssq = jnp.sum(out * out, axis=-1, keepdims=True)
    out_ref[...] = out / jnp.maximum(jnp.sqrt(ssq), 1e-12)


def _out_norm(parts, cnt_inv, yr, wl, bl):
    grid = _NODES // _BM
    return pl.pallas_call(
        _out_norm_body,
        grid=(grid,),
        in_specs=[
            pl.BlockSpec((1, _BM, _D), lambda i: (0, i, 0)),
            pl.BlockSpec((1, _BM, _D), lambda i: (1, i, 0)),
            pl.BlockSpec((_BM, 1), lambda i: (i, 0)),
            pl.BlockSpec((_BM, _D), lambda i: (i, 0)),
            pl.BlockSpec((_D, _D), lambda i: (0, 0)),
            pl.BlockSpec((1, _D), lambda i: (0, 0)),
        ],
        out_specs=pl.BlockSpec((_BM, _D), lambda i: (i, 0)),
        out_shape=jax.ShapeDtypeStruct((_NODES, _D), jnp.float32),
    )(parts, parts, cnt_inv, yr, wl, bl.reshape(1, _D))


_SC_MESH = plsc.VectorSubcoreMesh(core_axis_name="c", subcore_axis_name="s")
_SC_PARAMS = pltpu.CompilerParams(use_tc_tiling_on_sc=False,
                                  needs_layout_passes=False)


def _prep_body(src_hbm, dst_hbm, cnt_hbm, gsrc_hbm, gdst_hbm, nch_hbm,
               src_v, dst_v, hist_v, lsrc_v, ldst_v, cnts_v, cnt_sh, sem):
    c = lax.axis_index("c")
    s = lax.axis_index("s")
    wid = s * _NC + c
    pltpu.sync_copy(src_hbm.at[pl.ds(wid * _EPT, _EPT)], src_v.at[pl.ds(0, _EPT)])
    pltpu.sync_copy(dst_hbm.at[pl.ds(wid * _EPT, _EPT)], dst_v.at[pl.ds(0, _EPT)])

    zero16 = jnp.zeros((16,), jnp.float32)
    lanes = lax.iota(jnp.int32, 16)

    def zero_hist(t, carry):
        hist_v[t // 8, pl.ds((t % 8) * 16, 16)] = zero16
        return carry
    lax.fori_loop(0, _HISTR * 8, zero_hist, 0)

    @pl.when(s < 10)
    def _zero_sh():
        pltpu.sync_copy(hist_v.at[pl.ds(s * 8, 8)], cnt_sh.at[pl.ds(s * 8, 8)])
    plsc.subcore_barrier()

    ones = jnp.ones((16,), jnp.float32)

    def count(i, carry):
        nlanes = jnp.where(i == _NCHUNK, 8, 16)
        m = lanes < nlanes
        d = dst_v[pl.ds(i * 16, 16)]
        plsc.addupdate_scatter(hist_v, [lax.shift_right_logical(d, 7), d & 127],
                               ones, mask=m)
        return carry
    lax.fori_loop(0, _NCHUNK + 1, count, 0)

    for k in range(_HISTR // 16):
        idxv = lax.iota(jnp.int32, 16) + k * 16
        pltpu.sync_copy(hist_v.at[pl.ds(k * 16, 16)], cnt_sh.at[idxv], add=True)
    plsc.subcore_barrier()

    @pl.when(s < 10)
    def _writeback():
        pltpu.sync_copy(cnt_sh.at[pl.ds(s * 8, 8)], cnt_hbm.at[c, pl.ds(s * 8, 8)])

    def filt(i, ns):
        nlanes = jnp.where(i == _NCHUNK, 8, 16)
        m = lanes < nlanes
        d = dst_v[pl.ds(i * 16, 16)]
        sv = src_v[pl.ds(i * 16, 16)]
        out = []
        for q in range(_NBKT):
            inb = m & (d >= q * _BKT) & (d < (q + 1) * _BKT)
            plsc.store_compressed(lsrc_v.at[pl.ds(q * _CAP + ns[q], 16)], sv, mask=inb)
            plsc.store_compressed(ldst_v.at[pl.ds(q * _CAP + ns[q], 16)], d - q * _BKT,
                                  mask=inb)
            out.append(ns[q] + jnp.sum(inb.astype(jnp.int32)))
        return tuple(out)
    ns = lax.fori_loop(0, _NCHUNK + 1, filt, (0,) * _NBKT)

    trash16 = jnp.full((16,), _TRASH, jnp.int32)
    zero16i = jnp.zeros((16,), jnp.int32)
    for q in range(_NBKT):
        for t in range(5):
            lsrc_v[pl.ds(q * _CAP + ns[q] + t * 16, 16)] = zero16i
            ldst_v[pl.ds(q * _CAP + ns[q] + t * 16, 16)] = trash16
        nch = (ns[q] + _ROWS - 1) // _ROWS
        cnts_v[pl.ds(q * 16, 16)] = jnp.full((16,), nch, jnp.int32)
        pltpu.sync_copy(lsrc_v.at[pl.ds(q * _CAP, _CAP)], gsrc_hbm.at[wid, q])
        pltpu.sync_copy(ldst_v.at[pl.ds(q * _CAP, _CAP)], gdst_hbm.at[wid, q])
    pltpu.sync_copy(cnts_v, nch_hbm.at[wid])


def _sc_prep(src, dst):
    return pl.kernel(
        _prep_body,
        out_type=(
            jax.ShapeDtypeStruct((_NC, _HISTR, 128), jnp.float32),
            jax.ShapeDtypeStruct((_NW, _NBKT, _CAP), jnp.int32),
            jax.ShapeDtypeStruct((_NW, _NBKT, _CAP), jnp.int32),
            jax.ShapeDtypeStruct((_NW, _NBKT * 16), jnp.int32),
        ),
        mesh=_SC_MESH,
        compiler_params=_SC_PARAMS,
        scratch_types=[
            pltpu.VMEM((_EPT + 16,), jnp.int32),
            pltpu.VMEM((_EPT + 16,), jnp.int32),
            pltpu.VMEM((_HISTR, 128), jnp.float32),
            pltpu.VMEM((_NBKT * _CAP,), jnp.int32),
            pltpu.VMEM((_NBKT * _CAP,), jnp.int32),
            pltpu.VMEM((_NBKT * 16,), jnp.int32),
            pltpu.VMEM_SHARED((_HISTR, 128), jnp.float32),
            pltpu.SemaphoreType.DMA,
        ],
    )(src, dst)


def _segsum_body(xp_hbm, gsrc_hbm, gdst_hbm, nch_hbm, zrows_hbm, out_hbm,
                 gsrc_v, gdst_v, rows_a, rows_b, didx_a, didx_b,
                 zbuf_v, cbuf_v, acc_sh, sem_a, sem_b):
    c = lax.axis_index("c")
    s = lax.axis_index("s")
    wid = s * _NC + c
    pltpu.sync_copy(nch_hbm.at[wid], cbuf_v)
    pltpu.sync_copy(zrows_hbm, zbuf_v)

    lanes = lax.iota(jnp.int32, 16)

    for q in range(_NBKT):
        lo = q * _BKT

        for t in range(4):
            pltpu.sync_copy(zbuf_v, acc_sh.at[pl.ds(s * 64 + t * 16, 16)])
        pltpu.sync_copy(gsrc_hbm.at[wid, q], gsrc_v)
        pltpu.sync_copy(gdst_hbm.at[wid, q], gdst_v)
        nch = jnp.sum(cbuf_v[pl.ds(q * 16, 16)] * (lanes == 0).astype(jnp.int32))
        plsc.subcore_barrier()

        @pl.when(nch > 0)
        def _prime():
            pltpu.async_copy(xp_hbm.at[gsrc_v.at[pl.ds(0, _ROWS)]], rows_a, sem_a)

        def chunk(j, carry):
            nxt = j + 1

            @pl.when(j % 2 == 0)
            def _even():
                pltpu.make_async_copy(xp_hbm.at[gsrc_v.at[pl.ds(j * _ROWS, _ROWS)]],
                                      rows_a, sem_a).wait()

                @pl.when(nxt < nch)
                def _issue():
                    pltpu.async_copy(xp_hbm.at[gsrc_v.at[pl.ds(nxt * _ROWS, _ROWS)]],
                                     rows_b, sem_b)
                for k in range(_ROWS // 16):
                    didx_a[pl.ds(k * 16, 16)] = gdst_v[pl.ds(j * _ROWS + k * 16, 16)]
                pltpu.sync_copy(rows_a, acc_sh.at[didx_a], add=True)

            @pl.when(j % 2 == 1)
            def _odd():
                pltpu.make_async_copy(xp_hbm.at[gsrc_v.at[pl.ds(j * _ROWS, _ROWS)]],
                                      rows_b, sem_b).wait()

                @pl.when(nxt < nch)
                def _issue():
                    pltpu.async_copy(xp_hbm.at[gsrc_v.at[pl.ds(nxt * _ROWS, _ROWS)]],
                                     rows_a, sem_a)
                for k in range(_ROWS // 16):
                    didx_b[pl.ds(k * 16, 16)] = gdst_v[pl.ds(j * _ROWS + k * 16, 16)]
                pltpu.sync_copy(rows_b, acc_sh.at[didx_b], add=True)
            return carry
        lax.fori_loop(0, nch, chunk, 0)
        plsc.subcore_barrier()

        @pl.when(s < 15)
        def _wb():
            pltpu.sync_copy(acc_sh.at[pl.ds(s * 64, 64)],
                            out_hbm.at[c, pl.ds(lo + s * 64, 64)])

        @pl.when(s == 15)
        def _wb_tail():
            pltpu.sync_copy(acc_sh.at[pl.ds(960, 40)],
                            out_hbm.at[c, pl.ds(lo + 960, 40)])
        plsc.subcore_barrier()


def _sc_segsum(xp, gsrc, gdst, nch, zrows):
    return pl.kernel(
        _segsum_body,
        out_type=jax.ShapeDtypeStruct((_NC, _NODES, _D), jnp.float32),
        mesh=_SC_MESH,
        compiler_params=_SC_PARAMS,
        scratch_types=[
            pltpu.VMEM((_CAP,), jnp.int32),
            pltpu.VMEM((_CAP,), jnp.int32),
            pltpu.VMEM((_ROWS, _D), jnp.float32),
            pltpu.VMEM((_ROWS, _D), jnp.float32),
            pltpu.VMEM((_ROWS,), jnp.int32),
            pltpu.VMEM((_ROWS,), jnp.int32),
            pltpu.VMEM((16, _D), jnp.float32),
            pltpu.VMEM((_NBKT * 16,), jnp.int32),
            pltpu.VMEM_SHARED((_ACCR, _D), jnp.float32),
            pltpu.SemaphoreType.DMA,
            pltpu.SemaphoreType.DMA,
        ],
    )(xp, gsrc, gdst, nch, zrows)


def kernel(svg_path, svg_path_mask, edge_index, type_embed, coor_embed, W_in, b_in,
           W1p, b1p, W1l, b1l, W1r, W2p, b2p, W2l, b2l, W2r):
    svg = jnp.where(svg_path_mask, svg_path, 0)
    cmd_idx = svg[:, :, 0]
    coor_idx = svg[:, :, 1:] + 3
    idx_all = jnp.concatenate(
        [cmd_idx.reshape(_B, _N), coor_idx.reshape(_B, _N * (_C - 1))], axis=1
    ).reshape(_NODES)
    table = jnp.concatenate(
        [type_embed, coor_embed,
         jnp.zeros((_TPAD - 3 - 200, _D), jnp.float32)], axis=0
    )
    src = edge_index[0]
    dst = edge_index[1]
    zrows = jnp.zeros((16, _D), jnp.float32)

    cntp, gsrcw, gdstw, nchw = _sc_prep(src, dst)
    cnt = (cntp[0] + cntp[1]).reshape(_HISTR * 128)[:_NODES]
    cnt_inv = (1.0 / jnp.maximum(cnt, 1.0)).reshape(_NODES, 1)

    x = _embed_matmul(idx_all, table, W_in, b_in.reshape(1, _D))

    xp1 = _relu_matmul(x, W1p, b1p)
    yr1 = _matmul(xp1, W1r)
    parts1 = _sc_segsum(xp1, gsrcw, gdstw, nchw, zrows)
    x1 = _out_norm(parts1, cnt_inv, yr1, W1l, b1l)

    xp2 = _relu_matmul(x1, W2p, b2p)
    yr2 = _matmul(xp2, W2r)
    parts2 = _sc_segsum(xp2, gsrcw, gdstw, nchw, zrows)
    x2 = _out_norm(parts2, cnt_inv, yr2, W2l, b2l)
    return x2
